# baseline (speedup 1.0000x reference)
import functools

import jax
import jax.numpy as jnp
import numpy as np
from jax import lax
from jax.experimental import pallas as pl
from jax.experimental.pallas import tpu as pltpu
from jax.experimental.pallas import tpu_sc as plsc

_B, _N, _D = 32, 256, 4096
_T = _B * _N
_NC, _NS, _L = 2, 16, 16
_NW = _NC * _NS
_SPLIT = 3584
_T_SC = _T - _SPLIT
_RPW = _T_SC // _NW
_BLK = 256
_G = 4
_IDXC = 8 * (_RPW // _G + 2)
_LN2 = 0.6931471805599453

_INV23 = 1.0 / (1 << 23)


def _log2_times(t, x, gtab_v):
    xi = plsc.bitcast(x, jnp.int32)
    y = xi.astype(jnp.float32) * _INV23 + (-127.0)
    idx = lax.shift_right_logical(xi, 15) & 0xFF
    g = plsc.load_gather(gtab_v, [idx])
    return t * (y + g)


def _sc_body(ps_hbm, pt_hbm, mask_hbm, gtab_hbm, out_hbm,
             mask_v, idx_v, gtab_v,
             ps_b0, ps_b1, ps_b2, pt_b0, pt_b1, pt_b2, part_v,
             sem_ps0, sem_ps1, sem_ps2, sem_pt0, sem_pt1, sem_pt2):
    cid = lax.axis_index("c")
    sid = lax.axis_index("s")
    wid = sid * _NC + cid
    base = pl.multiple_of(_SPLIT + wid * _RPW, _RPW)

    pltpu.sync_copy(mask_hbm.at[pl.ds(base, _RPW)], mask_v)
    pltpu.sync_copy(gtab_hbm, gtab_v)

    lane = lax.iota(jnp.int32, _L)
    basev = jnp.zeros((_L,), jnp.int32) + base
    for i in range(_IDXC // _L):
        idx_v[pl.ds(i * _L, _L)] = basev
    off = jnp.int32(0)
    for i in range(_RPW // _L):
        mv = mask_v[pl.ds(i * _L, _L)]
        pos = plsc.cumsum(mv) + (off - 1)
        slot = lax.shift_left(lax.shift_right_logical(pos, 2), 3) | (pos & 3)
        rows = basev + (i * _L) + lane
        plsc.store_scatter(idx_v, [slot], rows, mask=mv > 0)
        off = off + jnp.sum(mv)
    local_n = off

    nch = (local_n + _G - 1) // _G
    zero16 = jnp.zeros((_L,), jnp.float32)
    ps_bufs = (ps_b0, ps_b1, ps_b2)
    pt_bufs = (pt_b0, pt_b1, pt_b2)
    ps_sems = (sem_ps0, sem_ps1, sem_ps2)
    pt_sems = (sem_pt0, sem_pt1, sem_pt2)
    _NB = 3

    def copies(j, b):
        idxs = idx_v.at[pl.ds(j * 8, _G)]
        return (pltpu.make_async_copy(ps_hbm.at[idxs], ps_bufs[b], ps_sems[b]),
                pltpu.make_async_copy(pt_hbm.at[idxs], pt_bufs[b], pt_sems[b]))

    for b in range(_NB - 1):
        @pl.when(b < nch)
        def _prime(b=b):
            for c in copies(b, b):
                c.start()

    def consume(j, b):
        for c in copies(j, b):
            c.wait()
        psb, ptb = ps_bufs[b], pt_bufs[b]

        @plsc.parallel_loop(0, _D, step=_L, unroll=2, carry=(zero16,) * _G)
        def accs(o, accs_in):
            return tuple(
                accs_in[r] + _log2_times(ptb[r, pl.ds(o, _L)],
                                         psb[r, pl.ds(o, _L)], gtab_v)
                for r in range(_G)
            )
        s = zero16
        for r in range(_G):
            s = s + jnp.where(j * _G + r < local_n, accs[r], zero16)
        return s

    ntrips = (nch + _NB - 1) // _NB

    def trip(t, acc):
        for b in range(_NB):
            j = t * _NB + b

            @pl.when(j + _NB - 1 < nch)
            def _prefetch(j=j, b=b):
                for c in copies(j + _NB - 1, (b + _NB - 1) % _NB):
                    c.start()

            acc = acc + lax.cond(j < nch,
                                 lambda j=j, b=b: consume(j, b),
                                 lambda: zero16)
        return acc

    acc = lax.fori_loop(0, ntrips, trip, zero16)

    part_v[pl.ds(0, _L)] = acc
    cntf = local_n.astype(jnp.float32)
    part_v[pl.ds(_L, _L)] = jnp.where(lane == 0, cntf, 0.0)
    pltpu.sync_copy(part_v, out_hbm.at[wid])


def _tc_dense_body(ps_ref, pt_ref, m_ref, out_ref, s_acc, c_acc):
    i = pl.program_id(0)

    @pl.when(i == 0)
    def _init():
        s_acc[0] = 0.0
        c_acc[0] = 0.0

    ps = ps_ref[...]
    pt = pt_ref[...]
    m = m_ref[...].astype(jnp.float32)
    per_tok = -(pt * jnp.log(ps)).sum(axis=-1)
    s_acc[0] += (per_tok * m).sum()
    c_acc[0] += m.sum()

    @pl.when(i == pl.num_programs(0) - 1)
    def _fin():
        out_ref[0, 0] = s_acc[0]
        out_ref[0, 1] = c_acc[0]


def _combine_body(parts_ref, tc_ref, out_ref):
    p = parts_ref[...]
    s = (-_LN2) * p[:, :_L].sum() + tc_ref[0, 0]
    c = p[:, _L:].sum() + tc_ref[0, 1]
    out_ref[0, 0] = s / jnp.maximum(c, 1.0)


def _gtab() -> jnp.ndarray:
    i = np.arange(256, dtype=np.float64)
    f = (i + 0.5) / 256.0
    return jnp.asarray(np.log2(1.0 + f) - f, dtype=jnp.float32)


def kernel(ps, pt, bool_masked_pos):
    ps2 = ps.reshape(_T, _D)
    pt2 = pt.reshape(_T, _D)
    mask = bool_masked_pos.reshape(_T).astype(jnp.int32)

    sc = pl.kernel(
        _sc_body,
        out_type=jax.ShapeDtypeStruct((_NW, 2 * _L), jnp.float32),
        mesh=plsc.VectorSubcoreMesh(core_axis_name="c", subcore_axis_name="s",
                                    num_cores=_NC, num_subcores=_NS),
        compiler_params=pltpu.CompilerParams(needs_layout_passes=False),
        scratch_types=[
            pltpu.VMEM((_RPW,), jnp.int32),
            pltpu.VMEM((_IDXC,), jnp.int32),
            pltpu.VMEM((256,), jnp.float32),
            pltpu.VMEM((_G, _D), jnp.float32),
            pltpu.VMEM((_G, _D), jnp.float32),
            pltpu.VMEM((_G, _D), jnp.float32),
            pltpu.VMEM((_G, _D), jnp.float32),
            pltpu.VMEM((_G, _D), jnp.float32),
            pltpu.VMEM((_G, _D), jnp.float32),
            pltpu.VMEM((2 * _L,), jnp.float32),
            pltpu.SemaphoreType.DMA,
            pltpu.SemaphoreType.DMA,
            pltpu.SemaphoreType.DMA,
            pltpu.SemaphoreType.DMA,
            pltpu.SemaphoreType.DMA,
            pltpu.SemaphoreType.DMA,
        ],
    )
    parts = sc(ps2, pt2, mask, _gtab())

    mask_b = bool_masked_pos.reshape(_T)
    tc_parts = pl.pallas_call(
        _tc_dense_body,
        grid=(_SPLIT // _BLK,),
        in_specs=[
            pl.BlockSpec((_BLK, _D), lambda i: (i, 0)),
            pl.BlockSpec((_BLK, _D), lambda i: (i, 0)),
            pl.BlockSpec((_BLK,), lambda i: (i,)),
        ],
        out_specs=pl.BlockSpec(memory_space=pltpu.SMEM),
        out_shape=jax.ShapeDtypeStruct((1, 2), jnp.float32),
        scratch_shapes=[
            pltpu.SMEM((1,), jnp.float32),
            pltpu.SMEM((1,), jnp.float32),
        ],
    )(ps2, pt2, mask_b)

    out = pl.pallas_call(
        _combine_body,
        in_specs=[
            pl.BlockSpec(memory_space=pltpu.VMEM),
            pl.BlockSpec(memory_space=pltpu.SMEM),
        ],
        out_specs=pl.BlockSpec(memory_space=pltpu.SMEM),
        out_shape=jax.ShapeDtypeStruct((1, 1), jnp.float32),
    )(parts, tc_parts)
    return out[0, 0]


# speedup vs baseline: 1.0156x; 1.0156x over previous
import jax
import jax.numpy as jnp
import numpy as np
from jax import lax
from jax.experimental import pallas as pl
from jax.experimental.pallas import tpu as pltpu
from jax.experimental.pallas import tpu_sc as plsc

_B, _N, _D = 32, 256, 4096
_T = _B * _N
_NC, _NS, _L = 2, 16, 16
_NW = _NC * _NS
_SPLIT = 3584
_T_SC = _T - _SPLIT
_RPW = _T_SC // _NW
_BLK = 256
_G = 4
_IDXC = 8 * (_RPW // _G + 2)
_LN2 = 0.6931471805599453

_INV23 = 1.0 / (1 << 23)


def _log2_times(t, x, gtab_v):
    xi = plsc.bitcast(x, jnp.int32)
    y = xi.astype(jnp.float32) * _INV23 + (-127.0)
    idx = lax.shift_right_logical(xi, 15) & 0xFF
    g = plsc.load_gather(gtab_v, [idx])
    return t * (y + g)


def _sc_body(ps_hbm, pt_hbm, mask_hbm, gtab_hbm, out_hbm,
             mask_v, idx_v, gtab_v,
             ps_b0, ps_b1, ps_b2, pt_b0, pt_b1, pt_b2, part_v,
             sem_ps0, sem_ps1, sem_ps2, sem_pt0, sem_pt1, sem_pt2):
    cid = lax.axis_index("c")
    sid = lax.axis_index("s")
    wid = sid * _NC + cid
    base = pl.multiple_of(_SPLIT + wid * _RPW, _RPW)

    pltpu.sync_copy(mask_hbm.at[pl.ds(base, _RPW)], mask_v)
    pltpu.sync_copy(gtab_hbm, gtab_v)

    lane = lax.iota(jnp.int32, _L)
    basev = jnp.zeros((_L,), jnp.int32) + base
    for i in range(_IDXC // _L):
        idx_v[pl.ds(i * _L, _L)] = basev
    off = jnp.int32(0)
    for i in range(_RPW // _L):
        mv = mask_v[pl.ds(i * _L, _L)]
        pos = plsc.cumsum(mv) + (off - 1)
        slot = lax.shift_left(lax.shift_right_logical(pos, 2), 3) | (pos & 3)
        rows = basev + (i * _L) + lane
        plsc.store_scatter(idx_v, [slot], rows, mask=mv > 0)
        off = off + jnp.sum(mv)
    local_n = off

    nch = (local_n + _G - 1) // _G
    zero16 = jnp.zeros((_L,), jnp.float32)
    ps_bufs = (ps_b0, ps_b1, ps_b2)
    pt_bufs = (pt_b0, pt_b1, pt_b2)
    ps_sems = (sem_ps0, sem_ps1, sem_ps2)
    pt_sems = (sem_pt0, sem_pt1, sem_pt2)
    _NB = 3

    def copies(j, b):
        idxs = idx_v.at[pl.ds(j * 8, _G)]
        return (pltpu.make_async_copy(ps_hbm.at[idxs], ps_bufs[b], ps_sems[b]),
                pltpu.make_async_copy(pt_hbm.at[idxs], pt_bufs[b], pt_sems[b]))

    for b in range(_NB - 1):
        @pl.when(b < nch)
        def _prime(b=b):
            for c in copies(b, b):
                c.start()

    def consume(j, b):
        for c in copies(j, b):
            c.wait()
        psb, ptb = ps_bufs[b], pt_bufs[b]

        @plsc.parallel_loop(0, _D, step=_L, unroll=2, carry=(zero16,) * _G)
        def accs(o, accs_in):
            return tuple(
                accs_in[r] + _log2_times(ptb[r, pl.ds(o, _L)],
                                         psb[r, pl.ds(o, _L)], gtab_v)
                for r in range(_G)
            )
        s = zero16
        for r in range(_G):
            s = s + jnp.where(j * _G + r < local_n, accs[r], zero16)
        return s

    ntrips = (nch + _NB - 1) // _NB

    def trip(t, acc):
        for b in range(_NB):
            j = t * _NB + b

            @pl.when(j + _NB - 1 < nch)
            def _prefetch(j=j, b=b):
                for c in copies(j + _NB - 1, (b + _NB - 1) % _NB):
                    c.start()

            acc = acc + lax.cond(j < nch,
                                 lambda j=j, b=b: consume(j, b),
                                 lambda: zero16)
        return acc

    acc = lax.fori_loop(0, ntrips, trip, zero16)

    part_v[pl.ds(0, _L)] = acc
    cntf = local_n.astype(jnp.float32)
    part_v[pl.ds(_L, _L)] = jnp.where(lane == 0, cntf, 0.0)
    pltpu.sync_copy(part_v, out_hbm.at[wid])


def _tc_dense_body(ps_ref, pt_ref, m_ref, out_ref, s_acc, c_acc):
    i = pl.program_id(0)

    @pl.when(i == 0)
    def _init():
        s_acc[0] = 0.0
        c_acc[0] = 0.0

    ps = ps_ref[...]
    pt = pt_ref[...]
    m = m_ref[...].astype(jnp.float32)
    per_tok = -(pt * jnp.log(ps)).sum(axis=-1)
    s_acc[0] += (per_tok * m).sum()
    c_acc[0] += m.sum()

    @pl.when(i == pl.num_programs(0) - 1)
    def _fin():
        out_ref[0, 0] = s_acc[0]
        out_ref[0, 1] = c_acc[0]


def _combine_body(parts_ref, tc_ref, out_ref):
    p = parts_ref[...]
    s = (-_LN2) * p[:, :_L].sum() + tc_ref[0, 0]
    c = p[:, _L:].sum() + tc_ref[0, 1]
    out_ref[0, 0] = s / jnp.maximum(c, 1.0)


def _gtab() -> jnp.ndarray:
    i = np.arange(256, dtype=np.float64)
    f = (i + 0.5) / 256.0
    return jnp.asarray(np.log2(1.0 + f) - f, dtype=jnp.float32)


def kernel(ps, pt, bool_masked_pos):
    ps2 = ps.reshape(_T, _D)
    pt2 = pt.reshape(_T, _D)
    mask = bool_masked_pos.reshape(_T).astype(jnp.int32)

    sc = pl.kernel(
        _sc_body,
        out_type=jax.ShapeDtypeStruct((_NW, 2 * _L), jnp.float32),
        mesh=plsc.VectorSubcoreMesh(core_axis_name="c", subcore_axis_name="s",
                                    num_cores=_NC, num_subcores=_NS),
        compiler_params=pltpu.CompilerParams(needs_layout_passes=False),
        scratch_types=[
            pltpu.VMEM((_RPW,), jnp.int32),
            pltpu.VMEM((_IDXC,), jnp.int32),
            pltpu.VMEM((256,), jnp.float32),
            pltpu.VMEM((_G, _D), jnp.float32),
            pltpu.VMEM((_G, _D), jnp.float32),
            pltpu.VMEM((_G, _D), jnp.float32),
            pltpu.VMEM((_G, _D), jnp.float32),
            pltpu.VMEM((_G, _D), jnp.float32),
            pltpu.VMEM((_G, _D), jnp.float32),
            pltpu.VMEM((2 * _L,), jnp.float32),
            pltpu.SemaphoreType.DMA,
            pltpu.SemaphoreType.DMA,
            pltpu.SemaphoreType.DMA,
            pltpu.SemaphoreType.DMA,
            pltpu.SemaphoreType.DMA,
            pltpu.SemaphoreType.DMA,
        ],
    )
    parts = sc(ps2, pt2, mask, _gtab())

    mask_b = bool_masked_pos.reshape(_T)
    tc_parts = pl.pallas_call(
        _tc_dense_body,
        grid=(_SPLIT // _BLK,),
        in_specs=[
            pl.BlockSpec((_BLK, _D), lambda i: (i, 0)),
            pl.BlockSpec((_BLK, _D), lambda i: (i, 0)),
            pl.BlockSpec((_BLK,), lambda i: (i,)),
        ],
        out_specs=pl.BlockSpec(memory_space=pltpu.SMEM),
        out_shape=jax.ShapeDtypeStruct((1, 2), jnp.float32),
        scratch_shapes=[
            pltpu.SMEM((1,), jnp.float32),
            pltpu.SMEM((1,), jnp.float32),
        ],
    )(ps2, pt2, mask_b)

    out = pl.pallas_call(
        _combine_body,
        in_specs=[
            pl.BlockSpec(memory_space=pltpu.VMEM),
            pl.BlockSpec(memory_space=pltpu.SMEM),
        ],
        out_specs=pl.BlockSpec(memory_space=pltpu.SMEM),
        out_shape=jax.ShapeDtypeStruct((1, 1), jnp.float32),
    )(parts, tc_parts)
    return out[0, 0]
